# baseline (speedup 1.0000x reference)
import math
import functools

import jax
import jax.numpy as jnp
from jax import lax
from jax.experimental import pallas as pl
from jax.experimental.pallas import tpu as pltpu
from jax.experimental.pallas import tpu_sc as plsc

BATCH = 4
SEQ_LEN = 2048
EMB = 128
NUM_WORKERS = 32
S_PER_W = SEQ_LEN // NUM_WORKERS
ROWS_PER_W = BATCH * S_PER_W
LANES = 16
KREG = EMB // LANES
SCALE = math.sqrt(EMB)


S_CHUNK = 32
N_SC = S_PER_W // S_CHUNK


def _body(x_hbm, tok_hbm, pos_hbm, out_hbm,
          idx_v, rows_v, pos_v, isem, gsem, osem, psem):
    c = lax.axis_index("c")
    s = lax.axis_index("s")
    wid = s * 2 + c
    s_base = wid * S_PER_W

    idx_copies = [
        pltpu.async_copy(
            x_hbm.at[b, pl.ds(s_base, S_PER_W)],
            idx_v.at[b],
            isem.at[b],
        )
        for b in range(BATCH)
    ]
    pos_copy = pltpu.async_copy(pos_hbm.at[pl.ds(s_base, S_PER_W)], pos_v, psem)

    gathers = {}

    def fire_gather(b, sc):
        off = b * S_PER_W + sc * S_CHUNK
        gathers[(b, sc)] = pltpu.async_copy(
            tok_hbm.at[idx_v.at[b, pl.ds(sc * S_CHUNK, S_CHUNK)]],
            rows_v.at[pl.ds(off, S_CHUNK)],
            gsem.at[b, sc],
        )

    for pair in range(BATCH // 2):
        b0 = pair * 2
        idx_copies[b0].wait()
        fire_gather(b0, 0)
        idx_copies[b0 + 1].wait()
        fire_gather(b0 + 1, 0)
        for sc in range(1, N_SC):
            fire_gather(b0, sc)
            fire_gather(b0 + 1, sc)
    pos_copy.wait()

    out_copies = []
    for pair in range(BATCH // 2):
        b0 = pair * 2
        for sc in range(N_SC):
            gathers[(b0, sc)].wait()
            gathers[(b0 + 1, sc)].wait()

            @plsc.parallel_loop(sc * S_CHUNK, (sc + 1) * S_CHUNK, unroll=2)
            def srow(i):
                p = [pos_v[i, pl.ds(k * LANES, LANES)] for k in range(KREG)]
                for b in (b0, b0 + 1):
                    r = b * S_PER_W + i
                    for k in range(KREG):
                        sl = pl.ds(k * LANES, LANES)
                        rows_v[r, sl] = (rows_v[r, sl] + p[k]) * SCALE

            for b in (b0, b0 + 1):
                out_copies.append(
                    pltpu.async_copy(
                        rows_v.at[pl.ds(b * S_PER_W + sc * S_CHUNK, S_CHUNK)],
                        out_hbm.at[b, pl.ds(s_base + sc * S_CHUNK, S_CHUNK)],
                        osem.at[b, sc],
                    )
                )
    for cp in out_copies:
        cp.wait()


def kernel(x_bs, tok_weight, pos_weight):
    mesh = plsc.VectorSubcoreMesh(core_axis_name="c", subcore_axis_name="s")
    run = functools.partial(
        pl.kernel,
        mesh=mesh,
        out_type=jax.ShapeDtypeStruct((BATCH, SEQ_LEN, EMB), jnp.float32),
        scratch_types=[
            pltpu.VMEM((BATCH, S_PER_W), jnp.int32),
            pltpu.VMEM((ROWS_PER_W, EMB), jnp.float32),
            pltpu.VMEM((S_PER_W, EMB), jnp.float32),
            pltpu.SemaphoreType.DMA((BATCH,)),
            pltpu.SemaphoreType.DMA((BATCH, N_SC)),
            pltpu.SemaphoreType.DMA((BATCH, N_SC)),
            pltpu.SemaphoreType.DMA,
        ],
    )(_body)

    return run(x_bs, tok_weight, pos_weight)


# speedup vs baseline: 1.0094x; 1.0094x over previous
import math
import functools

import jax
import jax.numpy as jnp
from jax import lax
from jax.experimental import pallas as pl
from jax.experimental.pallas import tpu as pltpu
from jax.experimental.pallas import tpu_sc as plsc

BATCH = 4
SEQ_LEN = 2048
EMB = 128
NUM_WORKERS = 32
S_PER_W = SEQ_LEN // NUM_WORKERS
ROWS_PER_W = BATCH * S_PER_W
LANES = 16
KREG = EMB // LANES
SCALE = math.sqrt(EMB)


S_CHUNK = 32
N_SC = S_PER_W // S_CHUNK


def _body(x_hbm, tok_hbm, pos_hbm, out_hbm,
          idx_v, rows_v, pos_v, isem, gsem, osem, psem):
    c = lax.axis_index("c")
    s = lax.axis_index("s")
    wid = s * 2 + c
    s_base = wid * S_PER_W

    idx_copies = [
        pltpu.async_copy(
            x_hbm.at[b, pl.ds(s_base, S_PER_W)],
            idx_v.at[b],
            isem.at[b],
        )
        for b in range(BATCH)
    ]
    pos_copy = pltpu.async_copy(pos_hbm.at[pl.ds(s_base, S_PER_W)], pos_v, psem)

    gathers = {}

    def fire_gather(b, sc):
        off = b * S_PER_W + sc * S_CHUNK
        gathers[(b, sc)] = pltpu.async_copy(
            tok_hbm.at[idx_v.at[b, pl.ds(sc * S_CHUNK, S_CHUNK)]],
            rows_v.at[pl.ds(off, S_CHUNK)],
            gsem.at[b, sc],
        )

    for pair in range(BATCH // 2):
        b0 = pair * 2
        idx_copies[b0].wait()
        fire_gather(b0, 0)
        idx_copies[b0 + 1].wait()
        fire_gather(b0 + 1, 0)
        for sc in range(1, N_SC):
            fire_gather(b0, sc)
            fire_gather(b0 + 1, sc)
    pos_copy.wait()

    out_copies = []
    for pair in range(BATCH // 2):
        b0 = pair * 2
        for sc in range(N_SC):
            gathers[(b0, sc)].wait()
            gathers[(b0 + 1, sc)].wait()

            @plsc.parallel_loop(sc * S_CHUNK, (sc + 1) * S_CHUNK)
            def srow(i):
                p = [pos_v[i, pl.ds(k * LANES, LANES)] for k in range(KREG)]
                for b in (b0, b0 + 1):
                    r = b * S_PER_W + i
                    for k in range(KREG):
                        sl = pl.ds(k * LANES, LANES)
                        rows_v[r, sl] = (rows_v[r, sl] + p[k]) * SCALE

            for b in (b0, b0 + 1):
                out_copies.append(
                    pltpu.async_copy(
                        rows_v.at[pl.ds(b * S_PER_W + sc * S_CHUNK, S_CHUNK)],
                        out_hbm.at[b, pl.ds(s_base + sc * S_CHUNK, S_CHUNK)],
                        osem.at[b, sc],
                    )
                )
    for cp in out_copies:
        cp.wait()


def kernel(x_bs, tok_weight, pos_weight):
    mesh = plsc.VectorSubcoreMesh(core_axis_name="c", subcore_axis_name="s")
    run = functools.partial(
        pl.kernel,
        mesh=mesh,
        out_type=jax.ShapeDtypeStruct((BATCH, SEQ_LEN, EMB), jnp.float32),
        scratch_types=[
            pltpu.VMEM((BATCH, S_PER_W), jnp.int32),
            pltpu.VMEM((ROWS_PER_W, EMB), jnp.float32),
            pltpu.VMEM((S_PER_W, EMB), jnp.float32),
            pltpu.SemaphoreType.DMA((BATCH,)),
            pltpu.SemaphoreType.DMA((BATCH, N_SC)),
            pltpu.SemaphoreType.DMA((BATCH, N_SC)),
            pltpu.SemaphoreType.DMA,
        ],
    )(_body)

    return run(x_bs, tok_weight, pos_weight)
